# baseline (speedup 1.0000x reference)
import functools

import jax
import jax.numpy as jnp
from jax import lax
from jax.experimental import pallas as pl
from jax.experimental.pallas import tpu as pltpu
from jax.experimental.pallas import tpu_sc as plsc

N = 10000
E = 320000
F = 128
G = 64

NC = 2
NS = 16
L = 16
NW = NC * NS

E_T = E // NW
K = 80
NCH = E_T // K
NPAD = 10240
HR_T = NPAD // NS
AR_T = NPAD // NS
NBUF = 4


def _hist_body(dst_r, za_hbm, ones_hbm, out_hbm, di_all, rows_v, acc, sems):
    c = lax.axis_index("c")
    s = lax.axis_index("s")
    wid = c * NS + s
    pltpu.sync_copy(ones_hbm, rows_v)
    pltpu.sync_copy(dst_r.at[wid], di_all)
    pltpu.sync_copy(za_hbm, acc.at[pl.ds(pl.multiple_of(s * AR_T, 8), AR_T)])
    plsc.subcore_barrier()

    def chunk(j, carry):
        pltpu.async_copy(rows_v, acc.at[di_all.at[j]], sems.at[j % NBUF], add=True)

        @pl.when(j >= NBUF)
        def _():
            pltpu.make_async_copy(
                rows_v, acc.at[di_all.at[j]], sems.at[j % NBUF]
            ).wait()

        return carry

    lax.fori_loop(0, NCH, chunk, 0)
    for b in range(NBUF):
        pltpu.make_async_copy(rows_v, acc.at[di_all.at[0]], sems.at[b]).wait()
    plsc.subcore_barrier()

    t0 = pl.multiple_of(s * AR_T, 8)
    co = pl.multiple_of(c * NPAD, 8)
    pltpu.sync_copy(acc.at[pl.ds(t0, AR_T)], out_hbm.at[pl.ds(co + t0, AR_T)])


def _agg_body(hp_hbm, src_hbm, dst_r, za_hbm, out_hbm, srd, di_all, rows3,
              acc, gsem, ssem, isem):
    c = lax.axis_index("c")
    s = lax.axis_index("s")
    wid = c * NS + s
    ebase = pl.multiple_of(wid * E_T, 8)
    pltpu.sync_copy(dst_r.at[wid], di_all)
    pltpu.sync_copy(src_hbm.at[pl.ds(ebase, K)], srd.at[0])
    pltpu.sync_copy(src_hbm.at[pl.ds(ebase + K, K)], srd.at[1])
    pltpu.async_copy(src_hbm.at[pl.ds(ebase + 2 * K, K)], srd.at[2], isem.at[2])
    pltpu.sync_copy(za_hbm, acc.at[pl.ds(pl.multiple_of(s * AR_T, 8), AR_T)])
    plsc.subcore_barrier()

    pltpu.async_copy(hp_hbm.at[srd.at[0]], rows3.at[0], gsem.at[0])
    pltpu.async_copy(hp_hbm.at[srd.at[1]], rows3.at[1], gsem.at[1])

    def chunk(j, carry):
        p = j % 3
        pn = (j + 2) % 3
        pltpu.make_async_copy(hp_hbm.at[srd.at[p]], rows3.at[p], gsem.at[p]).wait()

        @pl.when(j + 3 < NCH)
        def _():
            pltpu.async_copy(
                src_hbm.at[pl.ds(pl.multiple_of(ebase + (j + 3) * K, 8), K)],
                srd.at[p], isem.at[p],
            )

        pltpu.async_copy(rows3.at[p], acc.at[di_all.at[j]], ssem.at[p], add=True)

        @pl.when(j >= 1)
        def _():
            pltpu.make_async_copy(
                rows3.at[pn], acc.at[di_all.at[j]], ssem.at[pn]
            ).wait()

        @pl.when(j + 2 < NCH)
        def _():
            pltpu.make_async_copy(
                src_hbm.at[pl.ds(ebase, K)], srd.at[pn], isem.at[pn]
            ).wait()
            pltpu.async_copy(hp_hbm.at[srd.at[pn]], rows3.at[pn], gsem.at[pn])

        return carry

    lax.fori_loop(0, NCH, chunk, 0)
    pltpu.make_async_copy(
        rows3.at[0], acc.at[di_all.at[0]], ssem.at[(NCH - 1) % 3]
    ).wait()
    plsc.subcore_barrier()

    t0 = pl.multiple_of(s * AR_T, 8)
    co = pl.multiple_of(c * NPAD, 8)
    pltpu.sync_copy(acc.at[pl.ds(t0, AR_T)], out_hbm.at[pl.ds(co + t0, AR_T)])


@functools.lru_cache(maxsize=1)
def _sc_kernels():
    mesh = plsc.VectorSubcoreMesh(
        core_axis_name="c", subcore_axis_name="s", num_cores=NC, num_subcores=NS
    )
    hist = pl.kernel(
        _hist_body,
        out_type=jax.ShapeDtypeStruct((NC * NPAD, F), jnp.float32),
        mesh=mesh,
        scratch_types=[
            pltpu.VMEM((NCH, K), jnp.int32),
            pltpu.VMEM((K, F), jnp.float32),
            pltpu.VMEM_SHARED((NPAD, F), jnp.float32),
            pltpu.SemaphoreType.DMA((NBUF,)),
        ],
    )
    agg = pl.kernel(
        _agg_body,
        out_type=jax.ShapeDtypeStruct((NC * NPAD, F), jnp.float32),
        mesh=mesh,
        scratch_types=[
            pltpu.VMEM((3, K), jnp.int32),
            pltpu.VMEM((NCH, K), jnp.int32),
            pltpu.VMEM((3, K, F), jnp.float32),
            pltpu.VMEM_SHARED((NPAD, F), jnp.float32),
            pltpu.SemaphoreType.DMA((3,)),
            pltpu.SemaphoreType.DMA((3,)),
            pltpu.SemaphoreType.DMA((3,)),
        ],
    )
    return hist, agg


def _mm1_body(x_ref, w_ref, h_ref):
    h_ref[...] = jnp.dot(x_ref[...], w_ref[...], preferred_element_type=jnp.float32)


def _pre1_body(h_ref, deg_ref, h1p_ref, dinv_ref):
    deg = deg_ref[0:N, 0:1] + deg_ref[NPAD : NPAD + N, 0:1]
    dinv = lax.rsqrt(deg + 1.0)
    h1p_ref[...] = h_ref[...] * dinv
    dinv_ref[...] = dinv


def _mid_body(agg_ref, hp_ref, dinv_ref, b_ref, w_ref, out_ref):
    dinv = dinv_ref[...]
    hin = jnp.maximum(
        dinv * (agg_ref[0:N] + agg_ref[NPAD : NPAD + N] + hp_ref[...]) + b_ref[...],
        0.0,
    )
    out_ref[...] = (
        jnp.dot(hin, w_ref[...], preferred_element_type=jnp.float32) * dinv
    )


def _final_body(
    agg_ref, hp_ref, dinv_ref, b3_ref, batch_ref, wh1_ref, bh1_ref, wh2_ref,
    bh2_ref, out_ref,
):
    h3 = (
        dinv_ref[...] * (agg_ref[0:N] + agg_ref[NPAD : NPAD + N] + hp_ref[...])
        + b3_ref[...]
    )
    gids = lax.broadcasted_iota(jnp.int32, (G, N), 0)
    m = (gids == batch_ref[...]).astype(jnp.float32)
    cnt = jnp.sum(m, axis=1, keepdims=True)
    pooled = jnp.dot(m, h3, preferred_element_type=jnp.float32) / jnp.maximum(
        cnt, 1.0
    )
    z = jnp.maximum(
        jnp.dot(pooled, wh1_ref[...], preferred_element_type=jnp.float32)
        + bh1_ref[...],
        0.0,
    )
    out_ref[...] = (
        jnp.dot(z, wh2_ref[...], preferred_element_type=jnp.float32) + bh2_ref[...]
    )


def kernel(x, edge_index, batch, W1, b1, W2, b2, W3, b3, Wh1, bh1, Wh2, bh2):
    _hist, _agg = _sc_kernels()
    src_f = edge_index[0]
    dst_r = edge_index[1].reshape(NW, NCH, K)
    zeros_a = jnp.zeros((AR_T, F), jnp.float32)
    ones_rows = jnp.ones((K, F), jnp.float32)

    deg_arr = _hist(dst_r, zeros_a, ones_rows)

    hW1 = pl.pallas_call(
        _mm1_body, out_shape=jax.ShapeDtypeStruct((N, F), jnp.float32)
    )(x, W1)
    h1p, dinv = pl.pallas_call(
        _pre1_body,
        out_shape=(
            jax.ShapeDtypeStruct((N, F), jnp.float32),
            jax.ShapeDtypeStruct((N, 1), jnp.float32),
        ),
    )(hW1, deg_arr)

    agg1 = _agg(h1p, src_f, dst_r, zeros_a)
    h2p = pl.pallas_call(
        _mid_body, out_shape=jax.ShapeDtypeStruct((N, F), jnp.float32)
    )(agg1, h1p, dinv, b1.reshape(1, F), W2)

    agg2 = _agg(h2p, src_f, dst_r, zeros_a)
    h3p = pl.pallas_call(
        _mid_body, out_shape=jax.ShapeDtypeStruct((N, F), jnp.float32)
    )(agg2, h2p, dinv, b2.reshape(1, F), W3)

    agg3 = _agg(h3p, src_f, dst_r, zeros_a)
    out = pl.pallas_call(
        _final_body, out_shape=jax.ShapeDtypeStruct((G, F), jnp.float32)
    )(
        agg3, h3p, dinv, b3.reshape(1, F), batch.reshape(1, N),
        Wh1, bh1.reshape(1, F), Wh2, bh2.reshape(1, F),
    )
    return out


# speedup vs baseline: 1.0038x; 1.0038x over previous
import functools

import jax
import jax.numpy as jnp
from jax import lax
from jax.experimental import pallas as pl
from jax.experimental.pallas import tpu as pltpu
from jax.experimental.pallas import tpu_sc as plsc

N = 10000
E = 320000
F = 128
G = 64

NC = 2
NS = 16
L = 16
NW = NC * NS

E_T = E // NW
K = 80
NCH = E_T // K
NPAD = 10240
HR_T = NPAD // NS
AR_T = NPAD // NS
NBUF = 4


def _hist_body(dst_r, za_hbm, ones_hbm, out_hbm, di_all, rows_v, acc, sems):
    c = lax.axis_index("c")
    s = lax.axis_index("s")
    wid = c * NS + s
    pltpu.sync_copy(ones_hbm, rows_v)
    pltpu.sync_copy(dst_r.at[wid], di_all)
    pltpu.sync_copy(za_hbm, acc.at[pl.ds(pl.multiple_of(s * AR_T, 8), AR_T)])
    plsc.subcore_barrier()

    def chunk(j, carry):
        pltpu.async_copy(rows_v, acc.at[di_all.at[j]], sems.at[j % NBUF], add=True)

        @pl.when(j >= NBUF)
        def _():
            pltpu.make_async_copy(
                rows_v, acc.at[di_all.at[j]], sems.at[j % NBUF]
            ).wait()

        return carry

    lax.fori_loop(0, NCH, chunk, 0)
    for b in range(NBUF):
        pltpu.make_async_copy(rows_v, acc.at[di_all.at[0]], sems.at[b]).wait()
    plsc.subcore_barrier()

    t0 = pl.multiple_of(s * AR_T, 8)
    co = pl.multiple_of(c * NPAD, 8)
    pltpu.sync_copy(acc.at[pl.ds(t0, AR_T)], out_hbm.at[pl.ds(co + t0, AR_T)])


def _agg_body(hp_hbm, src_hbm, dst_r, za_hbm, out_hbm, srd, di_all, rows3,
              acc, gsem, ssem, isem):
    c = lax.axis_index("c")
    s = lax.axis_index("s")
    wid = c * NS + s
    ebase = pl.multiple_of(wid * E_T, 8)
    pltpu.sync_copy(dst_r.at[wid], di_all)
    pltpu.sync_copy(src_hbm.at[pl.ds(ebase, K)], srd.at[0])
    pltpu.sync_copy(src_hbm.at[pl.ds(ebase + K, K)], srd.at[1])
    pltpu.async_copy(src_hbm.at[pl.ds(ebase + 2 * K, K)], srd.at[2], isem.at[2])
    pltpu.sync_copy(za_hbm, acc.at[pl.ds(pl.multiple_of(s * AR_T, 8), AR_T)])
    plsc.subcore_barrier()

    pltpu.async_copy(hp_hbm.at[srd.at[0]], rows3.at[0], gsem.at[0])
    pltpu.async_copy(hp_hbm.at[srd.at[1]], rows3.at[1], gsem.at[1])

    def chunk(j, carry):
        p = j % 3
        pn = (j + 2) % 3
        pltpu.make_async_copy(hp_hbm.at[srd.at[p]], rows3.at[p], gsem.at[p]).wait()

        @pl.when(j + 3 < NCH)
        def _():
            pltpu.async_copy(
                src_hbm.at[pl.ds(pl.multiple_of(ebase + (j + 3) * K, 8), K)],
                srd.at[p], isem.at[p],
            )

        pltpu.async_copy(rows3.at[p], acc.at[di_all.at[j]], ssem.at[p], add=True)

        @pl.when(j >= 1)
        def _():
            pltpu.make_async_copy(
                rows3.at[pn], acc.at[di_all.at[j]], ssem.at[pn]
            ).wait()

        @pl.when(j + 2 < NCH)
        def _():
            pltpu.make_async_copy(
                src_hbm.at[pl.ds(ebase, K)], srd.at[pn], isem.at[pn]
            ).wait()
            pltpu.async_copy(hp_hbm.at[srd.at[pn]], rows3.at[pn], gsem.at[pn])

        return carry

    lax.fori_loop(0, NCH, chunk, 0)
    pltpu.make_async_copy(
        rows3.at[0], acc.at[di_all.at[0]], ssem.at[(NCH - 1) % 3]
    ).wait()
    plsc.subcore_barrier()

    t0 = pl.multiple_of(s * AR_T, 8)
    co = pl.multiple_of(c * NPAD, 8)
    pltpu.sync_copy(acc.at[pl.ds(t0, AR_T)], out_hbm.at[pl.ds(co + t0, AR_T)])


@functools.lru_cache(maxsize=1)
def _sc_kernels():
    mesh = plsc.VectorSubcoreMesh(
        core_axis_name="c", subcore_axis_name="s", num_cores=NC, num_subcores=NS
    )
    hist = pl.kernel(
        _hist_body,
        out_type=jax.ShapeDtypeStruct((NC * NPAD, F), jnp.float32),
        mesh=mesh,
        scratch_types=[
            pltpu.VMEM((NCH, K), jnp.int32),
            pltpu.VMEM((K, F), jnp.float32),
            pltpu.VMEM_SHARED((NPAD, F), jnp.float32),
            pltpu.SemaphoreType.DMA((NBUF,)),
        ],
    )
    agg = pl.kernel(
        _agg_body,
        out_type=jax.ShapeDtypeStruct((NC * NPAD, F), jnp.float32),
        mesh=mesh,
        scratch_types=[
            pltpu.VMEM((3, K), jnp.int32),
            pltpu.VMEM((NCH, K), jnp.int32),
            pltpu.VMEM((3, K, F), jnp.float32),
            pltpu.VMEM_SHARED((NPAD, F), jnp.float32),
            pltpu.SemaphoreType.DMA((3,)),
            pltpu.SemaphoreType.DMA((3,)),
            pltpu.SemaphoreType.DMA((3,)),
        ],
    )
    return hist, agg


def _pre1_body(x_ref, w_ref, deg_ref, h1p_ref, dinv_ref):
    deg = deg_ref[0:N, 0:1] + deg_ref[NPAD : NPAD + N, 0:1]
    dinv = lax.rsqrt(deg + 1.0)
    h = jnp.dot(x_ref[...], w_ref[...], preferred_element_type=jnp.float32)
    h1p_ref[...] = h * dinv
    dinv_ref[...] = dinv


def _mid_body(agg_ref, hp_ref, dinv_ref, b_ref, w_ref, out_ref):
    dinv = dinv_ref[...]
    hin = jnp.maximum(
        dinv * (agg_ref[0:N] + agg_ref[NPAD : NPAD + N] + hp_ref[...]) + b_ref[...],
        0.0,
    )
    out_ref[...] = (
        jnp.dot(hin, w_ref[...], preferred_element_type=jnp.float32) * dinv
    )


def _final_body(
    agg_ref, hp_ref, dinv_ref, b3_ref, batch_ref, wh1_ref, bh1_ref, wh2_ref,
    bh2_ref, out_ref,
):
    h3 = (
        dinv_ref[...] * (agg_ref[0:N] + agg_ref[NPAD : NPAD + N] + hp_ref[...])
        + b3_ref[...]
    )
    gids = lax.broadcasted_iota(jnp.int32, (G, N), 0)
    m = (gids == batch_ref[...]).astype(jnp.float32)
    cnt = jnp.sum(m, axis=1, keepdims=True)
    pooled = jnp.dot(m, h3, preferred_element_type=jnp.float32) / jnp.maximum(
        cnt, 1.0
    )
    z = jnp.maximum(
        jnp.dot(pooled, wh1_ref[...], preferred_element_type=jnp.float32)
        + bh1_ref[...],
        0.0,
    )
    out_ref[...] = (
        jnp.dot(z, wh2_ref[...], preferred_element_type=jnp.float32) + bh2_ref[...]
    )


def kernel(x, edge_index, batch, W1, b1, W2, b2, W3, b3, Wh1, bh1, Wh2, bh2):
    _hist, _agg = _sc_kernels()
    src_f = edge_index[0]
    dst_r = edge_index[1].reshape(NW, NCH, K)
    zeros_a = jnp.zeros((AR_T, F), jnp.float32)
    ones_rows = jnp.ones((K, F), jnp.float32)

    deg_arr = _hist(dst_r, zeros_a, ones_rows)

    h1p, dinv = pl.pallas_call(
        _pre1_body,
        out_shape=(
            jax.ShapeDtypeStruct((N, F), jnp.float32),
            jax.ShapeDtypeStruct((N, 1), jnp.float32),
        ),
    )(x, W1, deg_arr)

    agg1 = _agg(h1p, src_f, dst_r, zeros_a)
    h2p = pl.pallas_call(
        _mid_body, out_shape=jax.ShapeDtypeStruct((N, F), jnp.float32)
    )(agg1, h1p, dinv, b1.reshape(1, F), W2)

    agg2 = _agg(h2p, src_f, dst_r, zeros_a)
    h3p = pl.pallas_call(
        _mid_body, out_shape=jax.ShapeDtypeStruct((N, F), jnp.float32)
    )(agg2, h2p, dinv, b2.reshape(1, F), W3)

    agg3 = _agg(h3p, src_f, dst_r, zeros_a)
    out = pl.pallas_call(
        _final_body, out_shape=jax.ShapeDtypeStruct((G, F), jnp.float32)
    )(
        agg3, h3p, dinv, b3.reshape(1, F), batch.reshape(1, N),
        Wh1, bh1.reshape(1, F), Wh2, bh2.reshape(1, F),
    )
    return out


# speedup vs baseline: 1.0233x; 1.0194x over previous
import functools

import jax
import jax.numpy as jnp
from jax import lax
from jax.experimental import pallas as pl
from jax.experimental.pallas import tpu as pltpu
from jax.experimental.pallas import tpu_sc as plsc

N = 10000
E = 320000
F = 128
G = 64

NC = 2
NS = 16
L = 16
NW = NC * NS

E_T = E // NW
K = 80
NCH = E_T // K
NPAD = 10240
HR_T = NPAD // NS
AR_T = NPAD // NS
NBUF = 4


def _hist_body(dst_r, za_hbm, ones_hbm, out_hbm, di_all, rows_v, acc, sems):
    c = lax.axis_index("c")
    s = lax.axis_index("s")
    wid = c * NS + s
    zslice = acc.at[pl.ds(pl.multiple_of(s * AR_T, 8), AR_T)]
    pltpu.async_copy(ones_hbm, rows_v, sems.at[0])
    pltpu.async_copy(dst_r.at[wid], di_all, sems.at[1])
    pltpu.async_copy(za_hbm, zslice, sems.at[2])
    pltpu.make_async_copy(ones_hbm, rows_v, sems.at[0]).wait()
    pltpu.make_async_copy(dst_r.at[wid], di_all, sems.at[1]).wait()
    pltpu.make_async_copy(za_hbm, zslice, sems.at[2]).wait()
    plsc.subcore_barrier()

    def chunk(j, carry):
        pltpu.async_copy(rows_v, acc.at[di_all.at[j]], sems.at[j % NBUF], add=True)

        @pl.when(j >= NBUF)
        def _():
            pltpu.make_async_copy(
                rows_v, acc.at[di_all.at[j]], sems.at[j % NBUF]
            ).wait()

        return carry

    lax.fori_loop(0, NCH, chunk, 0)
    for b in range(NBUF):
        pltpu.make_async_copy(rows_v, acc.at[di_all.at[0]], sems.at[b]).wait()
    plsc.subcore_barrier()

    t0 = pl.multiple_of(s * AR_T, 8)
    co = pl.multiple_of(c * NPAD, 8)
    pltpu.sync_copy(acc.at[pl.ds(t0, AR_T)], out_hbm.at[pl.ds(co + t0, AR_T)])


def _agg_body(hp_hbm, src_hbm, dst_r, za_hbm, out_hbm, srd, di_all, rows3,
              acc, gsem, ssem, isem):
    c = lax.axis_index("c")
    s = lax.axis_index("s")
    wid = c * NS + s
    ebase = pl.multiple_of(wid * E_T, 8)
    zslice = acc.at[pl.ds(pl.multiple_of(s * AR_T, 8), AR_T)]
    pltpu.async_copy(dst_r.at[wid], di_all, gsem.at[0])
    pltpu.async_copy(src_hbm.at[pl.ds(ebase, K)], srd.at[0], isem.at[0])
    pltpu.async_copy(src_hbm.at[pl.ds(ebase + K, K)], srd.at[1], isem.at[1])
    pltpu.async_copy(src_hbm.at[pl.ds(ebase + 2 * K, K)], srd.at[2], isem.at[2])
    pltpu.async_copy(za_hbm, zslice, gsem.at[1])
    pltpu.make_async_copy(dst_r.at[wid], di_all, gsem.at[0]).wait()
    pltpu.make_async_copy(src_hbm.at[pl.ds(ebase, K)], srd.at[0], isem.at[0]).wait()
    pltpu.make_async_copy(src_hbm.at[pl.ds(ebase + K, K)], srd.at[1], isem.at[1]).wait()
    pltpu.make_async_copy(za_hbm, zslice, gsem.at[1]).wait()
    plsc.subcore_barrier()

    pltpu.async_copy(hp_hbm.at[srd.at[0]], rows3.at[0], gsem.at[0])
    pltpu.async_copy(hp_hbm.at[srd.at[1]], rows3.at[1], gsem.at[1])

    def chunk(j, carry):
        p = j % 3
        pn = (j + 2) % 3
        pltpu.make_async_copy(hp_hbm.at[srd.at[p]], rows3.at[p], gsem.at[p]).wait()

        @pl.when(j + 3 < NCH)
        def _():
            pltpu.async_copy(
                src_hbm.at[pl.ds(pl.multiple_of(ebase + (j + 3) * K, 8), K)],
                srd.at[p], isem.at[p],
            )

        pltpu.async_copy(rows3.at[p], acc.at[di_all.at[j]], ssem.at[p], add=True)

        @pl.when(j >= 1)
        def _():
            pltpu.make_async_copy(
                rows3.at[pn], acc.at[di_all.at[j]], ssem.at[pn]
            ).wait()

        @pl.when(j + 2 < NCH)
        def _():
            pltpu.make_async_copy(
                src_hbm.at[pl.ds(ebase, K)], srd.at[pn], isem.at[pn]
            ).wait()
            pltpu.async_copy(hp_hbm.at[srd.at[pn]], rows3.at[pn], gsem.at[pn])

        return carry

    lax.fori_loop(0, NCH, chunk, 0)
    pltpu.make_async_copy(
        rows3.at[0], acc.at[di_all.at[0]], ssem.at[(NCH - 1) % 3]
    ).wait()
    plsc.subcore_barrier()

    t0 = pl.multiple_of(s * AR_T, 8)
    co = pl.multiple_of(c * NPAD, 8)
    pltpu.sync_copy(acc.at[pl.ds(t0, AR_T)], out_hbm.at[pl.ds(co + t0, AR_T)])


@functools.lru_cache(maxsize=1)
def _sc_kernels():
    mesh = plsc.VectorSubcoreMesh(
        core_axis_name="c", subcore_axis_name="s", num_cores=NC, num_subcores=NS
    )
    hist = pl.kernel(
        _hist_body,
        out_type=jax.ShapeDtypeStruct((NC * NPAD, F), jnp.float32),
        mesh=mesh,
        scratch_types=[
            pltpu.VMEM((NCH, K), jnp.int32),
            pltpu.VMEM((K, F), jnp.float32),
            pltpu.VMEM_SHARED((NPAD, F), jnp.float32),
            pltpu.SemaphoreType.DMA((NBUF,)),
        ],
    )
    agg = pl.kernel(
        _agg_body,
        out_type=jax.ShapeDtypeStruct((NC * NPAD, F), jnp.float32),
        mesh=mesh,
        scratch_types=[
            pltpu.VMEM((3, K), jnp.int32),
            pltpu.VMEM((NCH, K), jnp.int32),
            pltpu.VMEM((3, K, F), jnp.float32),
            pltpu.VMEM_SHARED((NPAD, F), jnp.float32),
            pltpu.SemaphoreType.DMA((3,)),
            pltpu.SemaphoreType.DMA((3,)),
            pltpu.SemaphoreType.DMA((3,)),
        ],
    )
    return hist, agg


def _pre1_body(x_ref, w_ref, deg_ref, h1p_ref, dinv_ref):
    deg = deg_ref[0:N, 0:1] + deg_ref[NPAD : NPAD + N, 0:1]
    dinv = lax.rsqrt(deg + 1.0)
    h = jnp.dot(x_ref[...], w_ref[...], preferred_element_type=jnp.float32)
    h1p_ref[...] = h * dinv
    dinv_ref[...] = dinv


def _mid_body(agg_ref, hp_ref, dinv_ref, b_ref, w_ref, out_ref):
    dinv = dinv_ref[...]
    hin = jnp.maximum(
        dinv * (agg_ref[0:N] + agg_ref[NPAD : NPAD + N] + hp_ref[...]) + b_ref[...],
        0.0,
    )
    out_ref[...] = (
        jnp.dot(hin, w_ref[...], preferred_element_type=jnp.float32) * dinv
    )


def _final_body(
    agg_ref, hp_ref, dinv_ref, b3_ref, batch_ref, wh1_ref, bh1_ref, wh2_ref,
    bh2_ref, out_ref,
):
    h3 = (
        dinv_ref[...] * (agg_ref[0:N] + agg_ref[NPAD : NPAD + N] + hp_ref[...])
        + b3_ref[...]
    )
    gids = lax.broadcasted_iota(jnp.int32, (G, N), 0)
    m = (gids == batch_ref[...]).astype(jnp.float32)
    cnt = jnp.sum(m, axis=1, keepdims=True)
    pooled = jnp.dot(m, h3, preferred_element_type=jnp.float32) / jnp.maximum(
        cnt, 1.0
    )
    z = jnp.maximum(
        jnp.dot(pooled, wh1_ref[...], preferred_element_type=jnp.float32)
        + bh1_ref[...],
        0.0,
    )
    out_ref[...] = (
        jnp.dot(z, wh2_ref[...], preferred_element_type=jnp.float32) + bh2_ref[...]
    )


def kernel(x, edge_index, batch, W1, b1, W2, b2, W3, b3, Wh1, bh1, Wh2, bh2):
    _hist, _agg = _sc_kernels()
    src_f = edge_index[0]
    dst_r = edge_index[1].reshape(NW, NCH, K)
    zeros_a = jnp.zeros((AR_T, F), jnp.float32)
    ones_rows = jnp.ones((K, F), jnp.float32)

    deg_arr = _hist(dst_r, zeros_a, ones_rows)

    h1p, dinv = pl.pallas_call(
        _pre1_body,
        out_shape=(
            jax.ShapeDtypeStruct((N, F), jnp.float32),
            jax.ShapeDtypeStruct((N, 1), jnp.float32),
        ),
    )(x, W1, deg_arr)

    agg1 = _agg(h1p, src_f, dst_r, zeros_a)
    h2p = pl.pallas_call(
        _mid_body, out_shape=jax.ShapeDtypeStruct((N, F), jnp.float32)
    )(agg1, h1p, dinv, b1.reshape(1, F), W2)

    agg2 = _agg(h2p, src_f, dst_r, zeros_a)
    h3p = pl.pallas_call(
        _mid_body, out_shape=jax.ShapeDtypeStruct((N, F), jnp.float32)
    )(agg2, h2p, dinv, b2.reshape(1, F), W3)

    agg3 = _agg(h3p, src_f, dst_r, zeros_a)
    out = pl.pallas_call(
        _final_body, out_shape=jax.ShapeDtypeStruct((G, F), jnp.float32)
    )(
        agg3, h3p, dinv, b3.reshape(1, F), batch.reshape(1, N),
        Wh1, bh1.reshape(1, F), Wh2, bh2.reshape(1, F),
    )
    return out
